# baseline (speedup 1.0000x reference)
import functools

import jax
import jax.numpy as jnp
from jax import lax
from jax.experimental import pallas as pl
from jax.experimental.pallas import tpu as pltpu
from jax.experimental.pallas import tpu_sc as plsc

BATCH = 16384
EMB = 32
HID = 10
NC = 2
NS = 16
NW = NC * NS
BPW = BATCH // NW


def _gather_body(u_hbm, v_hbm, ue_hbm, ve_hbm, uout_hbm, vout_hbm,
                 idx_u, idx_v, rows_u, rows_v, sem_u, sem_v):
    wid = lax.axis_index("s") * NC + lax.axis_index("c")
    base = wid * BPW
    pltpu.sync_copy(u_hbm.at[pl.ds(base, BPW)], idx_u)
    cp_u = pltpu.async_copy(ue_hbm.at[idx_u], rows_u, sem_u)
    pltpu.sync_copy(v_hbm.at[pl.ds(base, BPW)], idx_v)
    cp_v = pltpu.async_copy(ve_hbm.at[idx_v], rows_v, sem_v)
    cp_u.wait()
    pltpu.sync_copy(rows_u, uout_hbm.at[pl.ds(base, BPW)])
    cp_v.wait()
    pltpu.sync_copy(rows_v, vout_hbm.at[pl.ds(base, BPW)])


@jax.jit
def _sc_gather(u, v, user_emb, item_emb):
    mesh = plsc.VectorSubcoreMesh(core_axis_name="c", subcore_axis_name="s")
    f = pl.kernel(
        _gather_body,
        out_type=[
            jax.ShapeDtypeStruct((BATCH, EMB), jnp.float32),
            jax.ShapeDtypeStruct((BATCH, EMB), jnp.float32),
        ],
        mesh=mesh,
        scratch_types=[
            pltpu.VMEM((BPW,), jnp.int32),
            pltpu.VMEM((BPW,), jnp.int32),
            pltpu.VMEM((BPW, EMB), jnp.float32),
            pltpu.VMEM((BPW, EMB), jnp.float32),
            pltpu.SemaphoreType.DMA,
            pltpu.SemaphoreType.DMA,
        ],
        compiler_params=pltpu.CompilerParams(use_tc_tiling_on_sc=False),
    )
    return f(u, v, user_emb, item_emb)


def _mlp_body(u_ref, v_ref, w1u_ref, w1v_ref, b1_ref, w2_ref, b2_ref, o_ref):
    U = jnp.maximum(u_ref[...], 0.0)
    V = jnp.maximum(v_ref[...], 0.0)
    h = jnp.dot(U, w1u_ref[...], preferred_element_type=jnp.float32)
    h += jnp.dot(V, w1v_ref[...], preferred_element_type=jnp.float32)
    h = jnp.maximum(h + b1_ref[...], 0.0)
    o_ref[...] = jnp.sum(h * w2_ref[...], axis=1, keepdims=True) + b2_ref[...]


@functools.partial(jax.jit, static_argnames=("bb",))
def _tc_mlp(urows, vrows, w1u, w1v, b1, w2, b2, bb=2048):
    grid = (BATCH // bb,)
    return pl.pallas_call(
        _mlp_body,
        grid=grid,
        in_specs=[
            pl.BlockSpec((bb, EMB), lambda i: (i, 0)),
            pl.BlockSpec((bb, EMB), lambda i: (i, 0)),
            pl.BlockSpec((EMB, HID), lambda i: (0, 0)),
            pl.BlockSpec((EMB, HID), lambda i: (0, 0)),
            pl.BlockSpec((1, HID), lambda i: (0, 0)),
            pl.BlockSpec((1, HID), lambda i: (0, 0)),
            pl.BlockSpec((1, 1), lambda i: (0, 0)),
        ],
        out_specs=pl.BlockSpec((bb, 1), lambda i: (i, 0)),
        out_shape=jax.ShapeDtypeStruct((BATCH, 1), jnp.float32),
    )(urows, vrows, w1u, w1v, b1, w2, b2)


def kernel(u, v, user_emb, item_emb, W1, b1, W2, b2):
    urows, vrows = _sc_gather(u, v, user_emb, item_emb)
    w1u = W1[:, :EMB].T
    w1v = W1[:, EMB:].T
    return _tc_mlp(urows, vrows, w1u, w1v,
                   b1.reshape(1, HID), W2.reshape(1, HID), b2.reshape(1, 1))


# speedup vs baseline: 2.2897x; 2.2897x over previous
import functools

import jax
import jax.numpy as jnp
from jax import lax
from jax.experimental import pallas as pl
from jax.experimental.pallas import tpu as pltpu
from jax.experimental.pallas import tpu_sc as plsc

BATCH = 16384
EMB = 32
HID = 10
NC = 2
NS = 16
NW = NC * NS
BPW = BATCH // NW
CH = 32
NCHUNK = BPW // CH
L = 16


def _fire_chunk(tbl, raw, gbuf, sem, base, k):
    for g in range(CH // L):
        vec = raw[pl.ds(k * CH + g * L, L)]
        for i in range(L):
            t = lax.shift_right_logical(vec[i], 3)
            pltpu.async_copy(tbl.at[pl.ds(t, 1)],
                             gbuf.at[pl.ds(g * L + i, 1)], sem)


def _extract_chunk(raw, gbuf, out_ref, k):
    lanes = lax.iota(jnp.int32, L)
    for g in range(CH // L):
        r = raw[pl.ds(k * CH + g * L, L)] & 7
        s = lanes + g * L
        for c in range(EMB):
            col = jnp.full((L,), c, jnp.int32)
            val = plsc.load_gather(gbuf, [s, r, col])
            out_ref[c, pl.ds(k * CH + g * L, L)] = jnp.maximum(val, 0.0)


def _gather_body(u_hbm, v_hbm, ue_hbm, ve_hbm, ut_hbm, vt_hbm,
                 raw_u, raw_v, gb0, gb1, out_u, out_v,
                 sem0, sem1, semo_u, semo_v):
    wid = lax.axis_index("s") * NC + lax.axis_index("c")
    base = wid * BPW
    pltpu.sync_copy(u_hbm.at[pl.ds(base, BPW)], raw_u)
    pltpu.sync_copy(v_hbm.at[pl.ds(base, BPW)], raw_v)

    for (tbl, raw, out_vmem, out_hbm, semo) in (
            (ue_hbm, raw_u, out_u, ut_hbm, semo_u),
            (ve_hbm, raw_v, out_v, vt_hbm, semo_v)):
        _fire_chunk(tbl, raw, gb0, sem0, base, 0)

        def chunk_pair(jj, carry, tbl=tbl, raw=raw, out_vmem=out_vmem):
            j0 = 2 * jj
            _fire_chunk(tbl, raw, gb1, sem1, base, j0 + 1)
            pltpu.make_async_copy(tbl.at[pl.ds(0, CH)], gb0, sem0).wait()
            _extract_chunk(raw, gb0, out_vmem, j0)

            @pl.when(j0 + 2 < NCHUNK)
            def _():
                _fire_chunk(tbl, raw, gb0, sem0, base, j0 + 2)

            pltpu.make_async_copy(tbl.at[pl.ds(0, CH)], gb1, sem1).wait()
            _extract_chunk(raw, gb1, out_vmem, j0 + 1)
            return carry

        lax.fori_loop(0, NCHUNK // 2, chunk_pair, None)
        pltpu.async_copy(out_vmem, out_hbm.at[:, pl.ds(base, BPW)], semo)

    pltpu.make_async_copy(out_u, ut_hbm.at[:, pl.ds(base, BPW)], semo_u).wait()
    pltpu.make_async_copy(out_v, vt_hbm.at[:, pl.ds(base, BPW)], semo_v).wait()


@jax.jit
def _sc_gather(u, v, ue3, ve3):
    mesh = plsc.VectorSubcoreMesh(core_axis_name="c", subcore_axis_name="s")
    f = pl.kernel(
        _gather_body,
        out_type=[
            jax.ShapeDtypeStruct((EMB, BATCH), jnp.float32),
            jax.ShapeDtypeStruct((EMB, BATCH), jnp.float32),
        ],
        mesh=mesh,
        scratch_types=[
            pltpu.VMEM((BPW,), jnp.int32),
            pltpu.VMEM((BPW,), jnp.int32),
            pltpu.VMEM((CH, 8, EMB), jnp.float32),
            pltpu.VMEM((CH, 8, EMB), jnp.float32),
            pltpu.VMEM((EMB, BPW), jnp.float32),
            pltpu.VMEM((EMB, BPW), jnp.float32),
            pltpu.SemaphoreType.DMA,
            pltpu.SemaphoreType.DMA,
            pltpu.SemaphoreType.DMA,
            pltpu.SemaphoreType.DMA,
        ],
        compiler_params=pltpu.CompilerParams(needs_layout_passes=False),
    )
    return f(u, v, ue3, ve3)


def _mlp_body(ut_ref, vt_ref, w1u_ref, w1v_ref, b1_ref, w2_ref, b2_ref, o_ref):
    h = jnp.dot(w1u_ref[...], ut_ref[...], preferred_element_type=jnp.float32)
    h += jnp.dot(w1v_ref[...], vt_ref[...], preferred_element_type=jnp.float32)
    h = jnp.maximum(h + b1_ref[...], 0.0)
    o_ref[...] = jnp.dot(w2_ref[...], h, preferred_element_type=jnp.float32) + b2_ref[...]


@functools.partial(jax.jit, static_argnames=("bb",))
def _tc_mlp(ut, vt, w1u, w1v, b1, w2, b2, bb=4096):
    grid = (BATCH // bb,)
    return pl.pallas_call(
        _mlp_body,
        grid=grid,
        in_specs=[
            pl.BlockSpec((EMB, bb), lambda i: (0, i)),
            pl.BlockSpec((EMB, bb), lambda i: (0, i)),
            pl.BlockSpec((HID, EMB), lambda i: (0, 0)),
            pl.BlockSpec((HID, EMB), lambda i: (0, 0)),
            pl.BlockSpec((HID, 1), lambda i: (0, 0)),
            pl.BlockSpec((1, HID), lambda i: (0, 0)),
            pl.BlockSpec((1, 1), lambda i: (0, 0)),
        ],
        out_specs=pl.BlockSpec((1, bb), lambda i: (0, i)),
        out_shape=jax.ShapeDtypeStruct((1, BATCH), jnp.float32),
    )(ut, vt, w1u, w1v, b1, w2, b2)


def kernel(u, v, user_emb, item_emb, W1, b1, W2, b2):
    u = u.astype(jnp.int32)
    v = v.astype(jnp.int32)
    ue3 = user_emb.reshape(user_emb.shape[0] // 8, 8, EMB)
    ve3 = item_emb.reshape(item_emb.shape[0] // 8, 8, EMB)
    ut, vt = _sc_gather(u, v, ue3, ve3)
    out_t = _tc_mlp(ut, vt, W1[:, :EMB], W1[:, EMB:],
                    b1.reshape(HID, 1), W2, b2.reshape(1, 1))
    return out_t.reshape(BATCH, 1)
